# Initial kernel scaffold; baseline (speedup 1.0000x reference)
import jax
import jax.numpy as jnp
from jax.experimental import pallas as pl


def kernel(x, adj, weight, self_weight):
    raise NotImplementedError("write your pallas kernel here")


# speedup vs baseline: 1.0878x; 1.0878x over previous
import functools

import jax
import jax.numpy as jnp
from jax import lax
from jax.experimental import pallas as pl
from jax.experimental.pallas import tpu as pltpu


def _ortho_kernel(w_ref, t_ref, *, gb: int, c: int, d: int, T: int,
                  eps: float, beta: float):
    g0 = pl.program_id(0) * gb
    W = w_ref[...].astype(jnp.float32)
    gi = lax.broadcasted_iota(jnp.int32, (gb, c, d), 0)
    ri = lax.broadcasted_iota(jnp.int32, (gb, c, d), 1)
    ci = lax.broadcasted_iota(jnp.int32, (gb, c, d), 2)
    eye_flat = (ci == (g0 + gi) * c + ri).astype(jnp.float32)
    Z = beta * W + (1.0 - beta) * eye_flat

    mean = jnp.sum(Z, axis=-1, keepdims=True) * (1.0 / d)
    Zc = Z - mean

    S = lax.dot_general(Zc, Zc, (((2,), (2,)), ((0,), (0,))),
                        preferred_element_type=jnp.float32)

    r_i = lax.broadcasted_iota(jnp.int32, (c, c), 0)
    c_i = lax.broadcasted_iota(jnp.int32, (c, c), 1)
    eye = (r_i == c_i).astype(jnp.float32)
    S = S + eps * eye[None, :, :]

    sumsq = jnp.sum(S * S, axis=(1, 2), keepdims=True)
    inv_norm = lax.rsqrt(sumsq)
    S = S * inv_norm

    dn = (((2,), (1,)), ((0,), (0,)))
    B = 1.5 * eye[None, :, :] - 0.5 * S
    for _ in range(T - 1):
        B2 = lax.dot_general(B, B, dn, preferred_element_type=jnp.float32)
        BS = lax.dot_general(B, S, dn, preferred_element_type=jnp.float32)
        B = 1.5 * B - 0.5 * lax.dot_general(B2, BS, dn,
                                            preferred_element_type=jnp.float32)
    B = B * jnp.sqrt(inv_norm)

    W_out = lax.dot_general(B, Zc, dn, preferred_element_type=jnp.float32)
    t_ref[...] = W_out.astype(t_ref.dtype)


def _main_kernel(adj_ref, x_ref, t_ref, sw_ref, out_ref, stats_ref, sup_ref,
                 *, tm: int):
    i = pl.program_id(0)
    xb = x_ref[...].astype(jnp.bfloat16)
    sup = jnp.dot(xb, t_ref[...], preferred_element_type=jnp.float32)
    sup_ref[...] = sup.astype(jnp.bfloat16)

    adjb = adj_ref[...].astype(jnp.bfloat16)
    acc = jnp.dot(adjb, sup_ref[...], preferred_element_type=jnp.float32)
    x_tile = x_ref[pl.ds(i * tm, tm), :].astype(jnp.bfloat16)
    acc = acc + jnp.dot(x_tile, sw_ref[...].astype(jnp.bfloat16),
                        preferred_element_type=jnp.float32)
    out_ref[...] = acc

    s = jnp.sum(acc, axis=0)
    sq = jnp.sum(acc * acc, axis=0)
    stats_ref[...] = jnp.stack([s, sq])[None, :, :]


def _bn_kernel(y_ref, stats_ref, o_ref, *, n: int, eps: float):
    st = stats_ref[...]
    mean = jnp.sum(st[:, 0, :], axis=0) * (1.0 / n)
    var = jnp.maximum(jnp.sum(st[:, 1, :], axis=0) * (1.0 / n) - mean * mean,
                      0.0)
    inv = lax.rsqrt(var + eps)
    o_ref[...] = (y_ref[...] - mean[None, :]) * inv[None, :]


def kernel(x, adj, weight, self_weight):
    beta, T, g, ortho_eps, bn_eps = 0.5, 5, 4, 1e-5, 1e-5
    n, f_in = x.shape
    f_out = weight.shape[1]
    c = f_in // g
    d = f_out

    vmem = pltpu.CompilerParams(dimension_semantics=("parallel",),
                                vmem_limit_bytes=56 * 1024 * 1024)

    gb = 2 if g % 2 == 0 else 1
    t3 = pl.pallas_call(
        functools.partial(_ortho_kernel, gb=gb, c=c, d=d, T=T,
                          eps=ortho_eps, beta=beta),
        out_shape=jax.ShapeDtypeStruct((g, c, d), jnp.bfloat16),
        grid=(g // gb,),
        in_specs=[pl.BlockSpec((gb, c, d), lambda i: (i, 0, 0))],
        out_specs=pl.BlockSpec((gb, c, d), lambda i: (i, 0, 0)),
        compiler_params=vmem,
    )(weight.reshape(g, c, d))
    t = t3.reshape(f_in, f_out)

    tm = min(512, n)
    n_tiles = n // tm
    out_pre, stats = pl.pallas_call(
        functools.partial(_main_kernel, tm=tm),
        out_shape=(jax.ShapeDtypeStruct((n, f_out), jnp.float32),
                   jax.ShapeDtypeStruct((n_tiles, 2, f_out), jnp.float32)),
        grid=(n_tiles,),
        in_specs=[pl.BlockSpec((tm, n), lambda i: (i, 0)),
                  pl.BlockSpec((n, f_in), lambda i: (0, 0)),
                  pl.BlockSpec((f_in, f_out), lambda i: (0, 0)),
                  pl.BlockSpec((f_in, f_out), lambda i: (0, 0))],
        out_specs=(pl.BlockSpec((tm, f_out), lambda i: (i, 0)),
                   pl.BlockSpec((1, 2, f_out), lambda i: (i, 0, 0))),
        scratch_shapes=[pltpu.VMEM((n, f_out), jnp.bfloat16)],
        compiler_params=vmem,
    )(adj, x, t, self_weight)

    tb = min(1024, n)
    out = pl.pallas_call(
        functools.partial(_bn_kernel, n=n, eps=bn_eps),
        out_shape=jax.ShapeDtypeStruct((n, f_out), x.dtype),
        grid=(n // tb,),
        in_specs=[pl.BlockSpec((tb, f_out), lambda i: (i, 0)),
                  pl.BlockSpec((n_tiles, 2, f_out), lambda i: (0, 0, 0))],
        out_specs=pl.BlockSpec((tb, f_out), lambda i: (i, 0)),
        compiler_params=vmem,
    )(out_pre, stats)
    return out


# speedup vs baseline: 1.1610x; 1.0673x over previous
import functools

import jax
import jax.numpy as jnp
from jax import lax
from jax.experimental import pallas as pl
from jax.experimental.pallas import tpu as pltpu


def _ortho_kernel(w_ref, t_ref, *, gb: int, c: int, d: int, T: int,
                  eps: float, beta: float):
    g0 = pl.program_id(0) * gb
    W = w_ref[...].astype(jnp.float32)
    gi = lax.broadcasted_iota(jnp.int32, (gb, c, d), 0)
    ri = lax.broadcasted_iota(jnp.int32, (gb, c, d), 1)
    ci = lax.broadcasted_iota(jnp.int32, (gb, c, d), 2)
    eye_flat = (ci == (g0 + gi) * c + ri).astype(jnp.float32)
    Z = beta * W + (1.0 - beta) * eye_flat

    mean = jnp.sum(Z, axis=-1, keepdims=True) * (1.0 / d)
    Zc = Z - mean

    S = lax.dot_general(Zc, Zc, (((2,), (2,)), ((0,), (0,))),
                        preferred_element_type=jnp.float32)

    r_i = lax.broadcasted_iota(jnp.int32, (c, c), 0)
    c_i = lax.broadcasted_iota(jnp.int32, (c, c), 1)
    eye = (r_i == c_i).astype(jnp.float32)
    S = S + eps * eye[None, :, :]

    sumsq = jnp.sum(S * S, axis=(1, 2), keepdims=True)
    inv_norm = lax.rsqrt(sumsq)
    S = S * inv_norm

    dn = (((2,), (1,)), ((0,), (0,)))
    B = 1.5 * eye[None, :, :] - 0.5 * S
    for _ in range(T - 1):
        B2 = lax.dot_general(B, B, dn, preferred_element_type=jnp.float32)
        BS = lax.dot_general(B, S, dn, preferred_element_type=jnp.float32)
        B = 1.5 * B - 0.5 * lax.dot_general(B2, BS, dn,
                                            preferred_element_type=jnp.float32)
    B = B * jnp.sqrt(inv_norm)

    W_out = lax.dot_general(B, Zc, dn, preferred_element_type=jnp.float32)
    t_ref[...] = W_out.astype(t_ref.dtype)


def _main_kernel(adj_ref, x_ref, t_ref, sw_ref, out_ref, stats_ref, sup_ref,
                 *, tm: int):
    i = pl.program_id(0)
    xb = x_ref[...].astype(jnp.bfloat16)
    sup = jnp.dot(xb, t_ref[...], preferred_element_type=jnp.float32)
    sup_ref[...] = sup.astype(jnp.bfloat16)

    adjb = adj_ref[...].astype(jnp.bfloat16)
    acc = jnp.dot(adjb, sup_ref[...], preferred_element_type=jnp.float32)
    x_tile = x_ref[pl.ds(i * tm, tm), :].astype(jnp.bfloat16)
    acc = acc + jnp.dot(x_tile, sw_ref[...].astype(jnp.bfloat16),
                        preferred_element_type=jnp.float32)
    out_ref[...] = acc.astype(out_ref.dtype)

    s = jnp.sum(acc, axis=0)
    sq = jnp.sum(acc * acc, axis=0)
    stats_ref[...] = jnp.stack([s, sq])[None, :, :]


def _bn_kernel(y_ref, stats_ref, o_ref, *, n: int, eps: float):
    st = stats_ref[...]
    mean = jnp.sum(st[:, 0, :], axis=0) * (1.0 / n)
    var = jnp.maximum(jnp.sum(st[:, 1, :], axis=0) * (1.0 / n) - mean * mean,
                      0.0)
    inv = lax.rsqrt(var + eps)
    y = y_ref[...].astype(jnp.float32)
    o_ref[...] = (y - mean[None, :]) * inv[None, :]


def kernel(x, adj, weight, self_weight):
    beta, T, g, ortho_eps, bn_eps = 0.5, 5, 4, 1e-5, 1e-5
    n, f_in = x.shape
    f_out = weight.shape[1]
    c = f_in // g
    d = f_out

    vmem = pltpu.CompilerParams(dimension_semantics=("parallel",),
                                vmem_limit_bytes=56 * 1024 * 1024)

    gb = 2 if g % 2 == 0 else 1
    t3 = pl.pallas_call(
        functools.partial(_ortho_kernel, gb=gb, c=c, d=d, T=T,
                          eps=ortho_eps, beta=beta),
        out_shape=jax.ShapeDtypeStruct((g, c, d), jnp.bfloat16),
        grid=(g // gb,),
        in_specs=[pl.BlockSpec((gb, c, d), lambda i: (i, 0, 0))],
        out_specs=pl.BlockSpec((gb, c, d), lambda i: (i, 0, 0)),
        compiler_params=vmem,
    )(weight.reshape(g, c, d))
    t = t3.reshape(f_in, f_out)

    tm = min(1024, n)
    n_tiles = n // tm
    out_pre, stats = pl.pallas_call(
        functools.partial(_main_kernel, tm=tm),
        out_shape=(jax.ShapeDtypeStruct((n, f_out), jnp.bfloat16),
                   jax.ShapeDtypeStruct((n_tiles, 2, f_out), jnp.float32)),
        grid=(n_tiles,),
        in_specs=[pl.BlockSpec((tm, n), lambda i: (i, 0)),
                  pl.BlockSpec((n, f_in), lambda i: (0, 0)),
                  pl.BlockSpec((f_in, f_out), lambda i: (0, 0)),
                  pl.BlockSpec((f_in, f_out), lambda i: (0, 0))],
        out_specs=(pl.BlockSpec((tm, f_out), lambda i: (i, 0)),
                   pl.BlockSpec((1, 2, f_out), lambda i: (i, 0, 0))),
        scratch_shapes=[pltpu.VMEM((n, f_out), jnp.bfloat16)],
        compiler_params=vmem,
    )(adj, x, t, self_weight)

    tb = min(2048, n)
    out = pl.pallas_call(
        functools.partial(_bn_kernel, n=n, eps=bn_eps),
        out_shape=jax.ShapeDtypeStruct((n, f_out), x.dtype),
        grid=(n // tb,),
        in_specs=[pl.BlockSpec((tb, f_out), lambda i: (i, 0)),
                  pl.BlockSpec((n_tiles, 2, f_out), lambda i: (0, 0, 0))],
        out_specs=pl.BlockSpec((tb, f_out), lambda i: (i, 0)),
        compiler_params=vmem,
    )(out_pre, stats)
    return out


# speedup vs baseline: 1.4537x; 1.2521x over previous
import functools

import jax
import jax.numpy as jnp
from jax import lax
from jax.experimental import pallas as pl
from jax.experimental.pallas import tpu as pltpu


def _compute_t(w_block, *, g: int, c: int, d: int, T: int, eps: float,
               beta: float):
    W = w_block.astype(jnp.float32)
    gi = lax.broadcasted_iota(jnp.int32, (g, c, d), 0)
    ri = lax.broadcasted_iota(jnp.int32, (g, c, d), 1)
    ci = lax.broadcasted_iota(jnp.int32, (g, c, d), 2)
    eye_flat = (ci == gi * c + ri).astype(jnp.float32)
    Z = beta * W + (1.0 - beta) * eye_flat

    mean = jnp.sum(Z, axis=-1, keepdims=True) * (1.0 / d)
    Zc = Z - mean

    S = lax.dot_general(Zc, Zc, (((2,), (2,)), ((0,), (0,))),
                        preferred_element_type=jnp.float32)

    r_i = lax.broadcasted_iota(jnp.int32, (c, c), 0)
    c_i = lax.broadcasted_iota(jnp.int32, (c, c), 1)
    eye = (r_i == c_i).astype(jnp.float32)
    S = S + eps * eye[None, :, :]

    sumsq = jnp.sum(S * S, axis=(1, 2), keepdims=True)
    inv_norm = lax.rsqrt(sumsq)
    S = S * inv_norm

    dn = (((2,), (1,)), ((0,), (0,)))
    B = 1.5 * eye[None, :, :] - 0.5 * S
    for _ in range(T - 1):
        B2 = lax.dot_general(B, B, dn, preferred_element_type=jnp.float32)
        BS = lax.dot_general(B, S, dn, preferred_element_type=jnp.float32)
        B = 1.5 * B - 0.5 * lax.dot_general(B2, BS, dn,
                                            preferred_element_type=jnp.float32)
    B = B * jnp.sqrt(inv_norm)
    return lax.dot_general(B, Zc, dn, preferred_element_type=jnp.float32)


def _fused_kernel(adj_ref, x_ref, w_ref, sw_ref, out_ref,
                  t_ref, sup_ref, stats_ref,
                  *, tm: int, n_tiles: int, g: int, c: int, d: int, T: int,
                  ortho_eps: float, beta: float, bn_eps: float):
    i = pl.program_id(0)
    n = x_ref.shape[0]

    @pl.when(i == 0)
    def _prep():
        t_full = _compute_t(w_ref[...], g=g, c=c, d=d, T=T, eps=ortho_eps,
                            beta=beta)
        for gg in range(g):
            t_ref[gg * c:(gg + 1) * c, :] = t_full[gg].astype(t_ref.dtype)
        xb = x_ref[...].astype(jnp.bfloat16)
        sup = jnp.dot(xb, t_ref[...], preferred_element_type=jnp.float32)
        sup_ref[...] = sup.astype(jnp.bfloat16)

    adjb = adj_ref[...].astype(jnp.bfloat16)
    acc = jnp.dot(adjb, sup_ref[...], preferred_element_type=jnp.float32)
    x_tile = x_ref[pl.ds(i * tm, tm), :].astype(jnp.bfloat16)
    acc = acc + jnp.dot(x_tile, sw_ref[...].astype(jnp.bfloat16),
                        preferred_element_type=jnp.float32)
    out_ref[pl.ds(i * tm, tm), :] = acc

    tile_stats = jnp.concatenate([jnp.sum(acc, axis=0, keepdims=True),
                                  jnp.sum(acc * acc, axis=0, keepdims=True)],
                                 axis=0)

    @pl.when(i == 0)
    def _init_stats():
        stats_ref[0:2, :] = tile_stats

    @pl.when(i > 0)
    def _acc_stats():
        stats_ref[0:2, :] = stats_ref[0:2, :] + tile_stats

    @pl.when(i == n_tiles - 1)
    def _normalize():
        mean = stats_ref[0, :] * (1.0 / n)
        var = jnp.maximum(stats_ref[1, :] * (1.0 / n) - mean * mean, 0.0)
        inv = lax.rsqrt(var + bn_eps)
        out_ref[...] = (out_ref[...] - mean[None, :]) * inv[None, :]


def kernel(x, adj, weight, self_weight):
    beta, T, g, ortho_eps, bn_eps = 0.5, 5, 4, 1e-5, 1e-5
    n, f_in = x.shape
    f_out = weight.shape[1]
    c = f_in // g
    d = f_out

    tm = min(512, n)
    n_tiles = n // tm

    out = pl.pallas_call(
        functools.partial(_fused_kernel, tm=tm, n_tiles=n_tiles, g=g, c=c,
                          d=d, T=T, ortho_eps=ortho_eps, beta=beta,
                          bn_eps=bn_eps),
        out_shape=jax.ShapeDtypeStruct((n, f_out), jnp.float32),
        grid=(n_tiles,),
        in_specs=[pl.BlockSpec((tm, n), lambda i: (i, 0)),
                  pl.BlockSpec((n, f_in), lambda i: (0, 0)),
                  pl.BlockSpec((g, c, d), lambda i: (0, 0, 0)),
                  pl.BlockSpec((f_in, f_out), lambda i: (0, 0))],
        out_specs=pl.BlockSpec((n, f_out), lambda i: (0, 0)),
        scratch_shapes=[pltpu.VMEM((f_in, f_out), jnp.bfloat16),
                        pltpu.VMEM((n, f_out), jnp.bfloat16),
                        pltpu.VMEM((8, f_out), jnp.float32)],
        compiler_params=pltpu.CompilerParams(
            dimension_semantics=("arbitrary",),
            vmem_limit_bytes=60 * 1024 * 1024),
    )(adj, x, weight.reshape(g, c, d), self_weight)
    return out


# speedup vs baseline: 1.5051x; 1.0354x over previous
import functools

import jax
import jax.numpy as jnp
from jax import lax
from jax.experimental import pallas as pl
from jax.experimental.pallas import tpu as pltpu


def _compute_t(w_block, *, g: int, c: int, d: int, T: int, eps: float,
               beta: float):
    W = w_block.astype(jnp.float32)
    gi = lax.broadcasted_iota(jnp.int32, (g, c, d), 0)
    ri = lax.broadcasted_iota(jnp.int32, (g, c, d), 1)
    ci = lax.broadcasted_iota(jnp.int32, (g, c, d), 2)
    eye_flat = (ci == gi * c + ri).astype(jnp.float32)
    Z = beta * W + (1.0 - beta) * eye_flat

    mean = jnp.sum(Z, axis=-1, keepdims=True) * (1.0 / d)
    Zc = Z - mean

    S = lax.dot_general(Zc, Zc, (((2,), (2,)), ((0,), (0,))),
                        preferred_element_type=jnp.float32)

    r_i = lax.broadcasted_iota(jnp.int32, (c, c), 0)
    c_i = lax.broadcasted_iota(jnp.int32, (c, c), 1)
    eye = (r_i == c_i).astype(jnp.float32)
    S = S + eps * eye[None, :, :]

    sumsq = jnp.sum(S * S, axis=(1, 2), keepdims=True)
    inv_norm = lax.rsqrt(sumsq)
    S = S * inv_norm

    dn = (((2,), (1,)), ((0,), (0,)))
    B = 1.5 * eye[None, :, :] - 0.5 * S
    for _ in range(T - 1):
        B2 = lax.dot_general(B, B, dn, preferred_element_type=jnp.float32)
        BS = lax.dot_general(B, S, dn, preferred_element_type=jnp.float32)
        B = 1.5 * B - 0.5 * lax.dot_general(B2, BS, dn,
                                            preferred_element_type=jnp.float32)
    B = B * jnp.sqrt(inv_norm)
    return lax.dot_general(B, Zc, dn, preferred_element_type=jnp.float32)


def _fused_kernel(adj_hbm, x_ref, w_ref, sw_ref, out_ref,
                  abuf, t_ref, sup_ref, stats_ref, in_sem,
                  *, tm: int, n_tiles: int, g: int, c: int, d: int, T: int,
                  ortho_eps: float, beta: float, bn_eps: float):
    n = x_ref.shape[0]

    def start_fetch(slot, step):
        pltpu.make_async_copy(adj_hbm.at[pl.ds(step * tm, tm), :],
                              abuf.at[slot], in_sem.at[slot]).start()

    def wait_fetch(slot):
        pltpu.make_async_copy(adj_hbm.at[pl.ds(0, tm), :],
                              abuf.at[slot], in_sem.at[slot]).wait()

    start_fetch(0, 0)
    if n_tiles > 1:
        start_fetch(1, 1)

    t_full = _compute_t(w_ref[...], g=g, c=c, d=d, T=T, eps=ortho_eps,
                        beta=beta)
    for gg in range(g):
        t_ref[gg * c:(gg + 1) * c, :] = t_full[gg].astype(t_ref.dtype)
    xb = x_ref[...].astype(jnp.bfloat16)
    sup = jnp.dot(xb, t_ref[...], preferred_element_type=jnp.float32)
    sup_ref[...] = sup.astype(jnp.bfloat16)
    stats_ref[...] = jnp.zeros_like(stats_ref)

    def tile_body(i, _):
        slot = lax.rem(i, 2)
        wait_fetch(slot)

        @pl.when(i + 2 < n_tiles)
        def _():
            start_fetch(slot, i + 2)

        adjb = abuf[slot].astype(jnp.bfloat16)
        acc = jnp.dot(adjb, sup_ref[...], preferred_element_type=jnp.float32)
        x_tile = x_ref[pl.ds(i * tm, tm), :].astype(jnp.bfloat16)
        acc = acc + jnp.dot(x_tile, sw_ref[...].astype(jnp.bfloat16),
                            preferred_element_type=jnp.float32)
        out_ref[pl.ds(i * tm, tm), :] = acc
        stats_ref[0:2, :] = stats_ref[0:2, :] + jnp.concatenate(
            [jnp.sum(acc, axis=0, keepdims=True),
             jnp.sum(acc * acc, axis=0, keepdims=True)], axis=0)
        return 0

    lax.fori_loop(0, n_tiles, tile_body, 0)

    mean = stats_ref[0, :] * (1.0 / n)
    var = jnp.maximum(stats_ref[1, :] * (1.0 / n) - mean * mean, 0.0)
    inv = lax.rsqrt(var + bn_eps)
    out_ref[...] = (out_ref[...] - mean[None, :]) * inv[None, :]


def kernel(x, adj, weight, self_weight):
    beta, T, g, ortho_eps, bn_eps = 0.5, 5, 4, 1e-5, 1e-5
    n, f_in = x.shape
    f_out = weight.shape[1]
    c = f_in // g
    d = f_out

    tm = min(512, n)
    n_tiles = n // tm

    out = pl.pallas_call(
        functools.partial(_fused_kernel, tm=tm, n_tiles=n_tiles, g=g, c=c,
                          d=d, T=T, ortho_eps=ortho_eps, beta=beta,
                          bn_eps=bn_eps),
        out_shape=jax.ShapeDtypeStruct((n, f_out), jnp.float32),
        grid=(1,),
        in_specs=[pl.BlockSpec(memory_space=pltpu.MemorySpace.HBM),
                  pl.BlockSpec((n, f_in), lambda i: (0, 0)),
                  pl.BlockSpec((g, c, d), lambda i: (0, 0, 0)),
                  pl.BlockSpec((f_in, f_out), lambda i: (0, 0))],
        out_specs=pl.BlockSpec((n, f_out), lambda i: (0, 0)),
        scratch_shapes=[pltpu.VMEM((2, tm, n), jnp.float32),
                        pltpu.VMEM((f_in, f_out), jnp.bfloat16),
                        pltpu.VMEM((n, f_out), jnp.bfloat16),
                        pltpu.VMEM((8, f_out), jnp.float32),
                        pltpu.SemaphoreType.DMA((2,))],
        compiler_params=pltpu.CompilerParams(
            dimension_semantics=("arbitrary",),
            vmem_limit_bytes=60 * 1024 * 1024),
    )(adj, x, weight.reshape(g, c, d), self_weight)
    return out
